# baseline (speedup 1.0000x reference)
import functools

import jax
import jax.numpy as jnp
from jax.experimental import pallas as pl
from jax.experimental.pallas import tpu as pltpu

B, T, D_IN = 32, 128, 1024
D_H1 = 512
H = 64
K = 1024
B_BLK = 8
N_BLK = B // B_BLK
RC = 256
LC = 128

_PREC = jax.lax.Precision.DEFAULT


def _dot(a, b, precision=_PREC):
    return jax.lax.dot_general(a, b, (((1,), (0,)), ((), ())),
                               precision=precision,
                               preferred_element_type=jnp.float32)


def _dott(a, b, precision=_PREC):
    return jax.lax.dot_general(a, b, (((1,), (1,)), ((), ())),
                               precision=precision,
                               preferred_element_type=jnp.float32)


def _fused_fwd_kernel(x_ref, w1_ref, b1_ref, w2_ref, b2_ref,
                      wck_ref, bc_ref, cb_ref,
                      tok_ref, q_ref, qt_ref, ft_ref, commit_ref,
                      m_scr):
    rows = B_BLK * T

    xb = x_ref[...].reshape(rows, D_IN)
    h1 = jnp.maximum(_dott(xb, w1_ref[...]) + b1_ref[...], 0.0)
    f = jnp.maximum(_dott(h1, w2_ref[...]) + b2_ref[...], 0.0)

    f3 = f.reshape(B_BLK, T, H)
    zpad = jnp.zeros((B_BLK, 1, H), dtype=jnp.float32)
    fprev = jnp.concatenate([zpad, f3[:, :-1, :]], axis=1).reshape(rows, H)
    fnext = jnp.concatenate([f3[:, 1:, :], zpad], axis=1).reshape(rows, H)
    fcat = jnp.concatenate([fprev, f, fnext], axis=1)
    feats = _dott(fcat, wck_ref[...]) + bc_ref[...]

    cb = cb_ref[...]
    a2 = jnp.sum(feats * feats, axis=1, keepdims=True)
    cb2 = _dott(jnp.ones((1, H), jnp.float32), cb * cb)
    m_scr[...] = _dott(feats, cb)

    idx_chunks = []
    for r in range(rows // RC):
        a2r = a2[r * RC:(r + 1) * RC]
        dmin_r = jnp.full((RC, 1), jnp.inf, jnp.float32)
        dist_tiles = []
        for c in range(K // LC):
            mc = m_scr[r * RC:(r + 1) * RC, c * LC:(c + 1) * LC]
            d2c = (a2r - 2.0 * mc) + cb2[:, c * LC:(c + 1) * LC]
            distc = jnp.sqrt(jnp.maximum(d2c, 0.0))
            dist_tiles.append(distc)
            dmin_r = jnp.minimum(
                dmin_r, jnp.min(distc, axis=1, keepdims=True))
        idx_r = jnp.full((RC, 1), K, jnp.int32)
        for c in range(K // LC):
            lanec = (jax.lax.broadcasted_iota(jnp.int32, (RC, LC), 1)
                     + c * LC)
            cand = jnp.where(dist_tiles[c] <= dmin_r, lanec, K)
            idx_r = jnp.minimum(
                idx_r, jnp.min(cand, axis=1, keepdims=True))
        idx_chunks.append(idx_r)
    idx = jnp.concatenate(idx_chunks, axis=0)

    tok_ref[...] = idx.reshape(B_BLK, T)

    onehot = (jax.lax.broadcasted_iota(jnp.int32, (rows, K), 1)
              == idx).astype(jnp.float32)
    q = _dot(onehot, cb)
    q3 = q.reshape(B_BLK, T, H)
    q_ref[...] = q3
    qt_ref[...] = jnp.swapaxes(q3, 0, 1)
    ft_ref[...] = jnp.swapaxes(feats.reshape(B_BLK, T, H), 0, 1)

    dq = feats - q
    commit_ref[...] = jnp.sum(dq * dq, axis=(0, 1),
                              keepdims=True).reshape(1, 1, 1)


def _gru_kernel(q_ref, f_ref, wicat_ref, bicat_ref, whcat_ref, bhcat_ref,
                ctx_ref, gi_ref):
    Tm = T - 1
    rows = Tm * B
    qflat = q_ref[0:Tm].reshape(rows, H)
    gi_ref[...] = (_dot(qflat, wicat_ref[...])
                   + bicat_ref[...]).reshape(Tm, B, 384)

    whcat = whcat_ref[...]
    bhcat = bhcat_ref[...]

    def step(t, carry):
        h, acc = carry
        gh = _dot(h, whcat) + bhcat
        gi = gi_ref[pl.ds(t, 1)].reshape(B, 384)
        r = jax.nn.sigmoid(gi[:, 0:H] + gh[:, 0:H])
        z = jax.nn.sigmoid(gi[:, 128:128 + H] + gh[:, 128:128 + H])
        n = jnp.tanh(gi[:, 256:256 + H] + r * gh[:, 256:256 + H])
        h_new = (1.0 - z) * n + z * h
        ft = f_ref[pl.ds(t + 1, 1)].reshape(B, H)
        d = h_new - ft
        return h_new, acc + d * d

    h0 = jnp.zeros((B, H), dtype=jnp.float32)
    acc0 = jnp.zeros((B, H), dtype=jnp.float32)
    _, acc = jax.lax.fori_loop(0, Tm, step, (h0, acc0), unroll=4)
    ctx_ref[...] = jnp.sum(acc, axis=(0, 1), keepdims=True) / (B * Tm * H)


@functools.partial(jax.jit, static_argnames=())
def kernel(x, W1, b1, W2, b2, Wc, bc, codebook, W_ih, W_hh, b_ih, b_hh):
    wck = Wc.transpose(0, 2, 1).reshape(H, 3 * H)

    full = lambda shp: pl.BlockSpec(shp, lambda i: (0,) * len(shp))
    tok, quantized, qT, fT, commit = pl.pallas_call(
        _fused_fwd_kernel,
        grid=(N_BLK,),
        in_specs=[
            pl.BlockSpec((B_BLK, T, D_IN), lambda i: (i, 0, 0)),
            full((D_H1, D_IN)), full((1, D_H1)),
            full((H, D_H1)), full((1, H)),
            full((H, 3 * H)), full((1, H)),
            full((K, H)),
        ],
        out_specs=[
            pl.BlockSpec((B_BLK, T), lambda i: (i, 0)),
            pl.BlockSpec((B_BLK, T, H), lambda i: (i, 0, 0)),
            pl.BlockSpec((T, B_BLK, H), lambda i: (0, i, 0)),
            pl.BlockSpec((T, B_BLK, H), lambda i: (0, i, 0)),
            pl.BlockSpec((1, 1, 1), lambda i: (i, 0, 0)),
        ],
        out_shape=[
            jax.ShapeDtypeStruct((B, T), jnp.int32),
            jax.ShapeDtypeStruct((B, T, H), jnp.float32),
            jax.ShapeDtypeStruct((T, B, H), jnp.float32),
            jax.ShapeDtypeStruct((T, B, H), jnp.float32),
            jax.ShapeDtypeStruct((N_BLK, 1, 1), jnp.float32),
        ],
        scratch_shapes=[pltpu.VMEM((B_BLK * T, K), jnp.float32)],
        compiler_params=pltpu.CompilerParams(
            dimension_semantics=("parallel",)),
    )(x, W1, b1.reshape(1, D_H1), W2, b2.reshape(1, H), wck,
      bc.reshape(1, H), codebook)

    zpadw = jnp.zeros((H, 128 - H), jnp.float32)
    wicat = jnp.concatenate(
        [W_ih[0:H].T, zpadw, W_ih[H:2 * H].T, zpadw, W_ih[2 * H:3 * H].T,
         zpadw], axis=1)
    whcat = jnp.concatenate(
        [W_hh[0:H].T, zpadw, W_hh[H:2 * H].T, zpadw, W_hh[2 * H:3 * H].T,
         zpadw], axis=1)
    zpadb = jnp.zeros((1, 128 - H), jnp.float32)
    bicat = jnp.concatenate(
        [b_ih[0:H].reshape(1, H), zpadb, b_ih[H:2 * H].reshape(1, H), zpadb,
         b_ih[2 * H:3 * H].reshape(1, H), zpadb], axis=1)
    bhcat = jnp.concatenate(
        [b_hh[0:H].reshape(1, H), zpadb, b_hh[H:2 * H].reshape(1, H), zpadb,
         b_hh[2 * H:3 * H].reshape(1, H), zpadb], axis=1)

    ctx = pl.pallas_call(
        _gru_kernel,
        scratch_shapes=[
            pltpu.VMEM((T - 1, B, 384), jnp.float32),
        ],
        out_shape=jax.ShapeDtypeStruct((1, 1), jnp.float32),
    )(qT, fT, wicat, bicat, whcat, bhcat)

    commitment_loss = jnp.sum(commit) / (B * T * H)
    codebook_loss = commitment_loss
    context_loss = ctx[0, 0]
    vq_loss = commitment_loss + 0.25 * codebook_loss + 0.1 * context_loss
    return (tok, quantized, commitment_loss, codebook_loss,
            context_loss, vq_loss)


# speedup vs baseline: 1.0703x; 1.0703x over previous
import functools

import jax
import jax.numpy as jnp
from jax.experimental import pallas as pl
from jax.experimental.pallas import tpu as pltpu

B, T, D_IN = 32, 128, 1024
D_H1 = 512
H = 64
K = 1024
B_BLK = 8
N_BLK = B // B_BLK
G = 384

_PREC = jax.lax.Precision.DEFAULT


def _dot(a, b, precision=_PREC):
    return jax.lax.dot_general(a, b, (((1,), (0,)), ((), ())),
                               precision=precision,
                               preferred_element_type=jnp.float32)


def _dott(a, b, precision=_PREC):
    return jax.lax.dot_general(a, b, (((1,), (1,)), ((), ())),
                               precision=precision,
                               preferred_element_type=jnp.float32)


def _vq_fused_kernel(x_ref, w1_ref, b1_ref, w2_ref, b2_ref,
                     wck_ref, bc_ref, cb_ref, wicat_ref, bicat_ref,
                     whcat_ref, bhcat_ref,
                     tok_ref, q_ref, commit_ref, ctx_ref,
                     gi_scr, ft_scr):
    i = pl.program_id(0)
    rows = B_BLK * T

    xb = x_ref[...].reshape(rows, D_IN)
    h1 = jnp.maximum(_dott(xb, w1_ref[...]) + b1_ref[...], 0.0)
    f = jnp.maximum(_dott(h1, w2_ref[...]) + b2_ref[...], 0.0)

    f3 = f.reshape(B_BLK, T, H)
    zpad = jnp.zeros((B_BLK, 1, H), dtype=jnp.float32)
    fprev = jnp.concatenate([zpad, f3[:, :-1, :]], axis=1).reshape(rows, H)
    fnext = jnp.concatenate([f3[:, 1:, :], zpad], axis=1).reshape(rows, H)
    fcat = jnp.concatenate([fprev, f, fnext], axis=1)
    feats = _dott(fcat, wck_ref[...]) + bc_ref[...]

    cb = cb_ref[...]
    a2 = jnp.sum(feats * feats, axis=1, keepdims=True)
    m = _dott(feats, cb)
    cb2 = _dott(jnp.ones((1, H), jnp.float32), cb * cb)
    d2 = (a2 - 2.0 * m) + cb2
    dist = jnp.sqrt(jnp.maximum(d2, 0.0))
    dmin = jnp.min(dist, axis=1, keepdims=True)
    lane = jax.lax.broadcasted_iota(jnp.int32, (rows, K), 1)
    idx = jnp.min(jnp.where(dist <= dmin, lane, K), axis=1)

    tok_ref[...] = idx.reshape(B_BLK, T)

    onehot = (jax.lax.broadcasted_iota(jnp.int32, (rows, K), 1)
              == idx[:, None]).astype(jnp.float32)
    q = _dot(onehot, cb)
    q3 = q.reshape(B_BLK, T, H)
    q_ref[...] = q3

    dq = feats - q
    commit_ref[...] = jnp.sum(dq * dq, axis=(0, 1),
                              keepdims=True).reshape(1, 1, 1)

    Tm = T - 1
    qt = jnp.swapaxes(q3, 0, 1)
    gi = (_dot(qt[0:Tm].reshape(Tm * B_BLK, H), wicat_ref[...])
          + bicat_ref[...])
    gi_scr[:, pl.ds(i * B_BLK, B_BLK), :] = gi.reshape(Tm, B_BLK, G)
    ft_scr[:, pl.ds(i * B_BLK, B_BLK), :] = (
        jnp.swapaxes(feats.reshape(B_BLK, T, H), 0, 1))

    @pl.when(i == N_BLK - 1)
    def _gru_tail():
        whcat = whcat_ref[...]
        bhcat = bhcat_ref[...]

        def step(t, carry):
            h, acc = carry
            gh = _dot(h, whcat) + bhcat
            gi_t = gi_scr[pl.ds(t, 1)].reshape(B, G)
            r = jax.nn.sigmoid(gi_t[:, 0:H] + gh[:, 0:H])
            z = jax.nn.sigmoid(gi_t[:, 128:128 + H] + gh[:, 128:128 + H])
            n = jnp.tanh(gi_t[:, 256:256 + H] + r * gh[:, 256:256 + H])
            h_new = (1.0 - z) * n + z * h
            ft = ft_scr[pl.ds(t + 1, 1)].reshape(B, H)
            d = h_new - ft
            return h_new, acc + d * d

        h0 = jnp.zeros((B, H), dtype=jnp.float32)
        acc0 = jnp.zeros((B, H), dtype=jnp.float32)
        _, acc = jax.lax.fori_loop(0, Tm, step, (h0, acc0), unroll=4)
        ctx_ref[...] = (jnp.sum(acc, axis=(0, 1), keepdims=True)
                        / (B * Tm * H))


@functools.partial(jax.jit, static_argnames=())
def kernel(x, W1, b1, W2, b2, Wc, bc, codebook, W_ih, W_hh, b_ih, b_hh):
    wck = Wc.transpose(0, 2, 1).reshape(H, 3 * H)

    zpadw = jnp.zeros((H, 128 - H), jnp.float32)
    wicat = jnp.concatenate(
        [W_ih[0:H].T, zpadw, W_ih[H:2 * H].T, zpadw, W_ih[2 * H:3 * H].T,
         zpadw], axis=1)
    whcat = jnp.concatenate(
        [W_hh[0:H].T, zpadw, W_hh[H:2 * H].T, zpadw, W_hh[2 * H:3 * H].T,
         zpadw], axis=1)
    zpadb = jnp.zeros((1, 128 - H), jnp.float32)
    bicat = jnp.concatenate(
        [b_ih[0:H].reshape(1, H), zpadb, b_ih[H:2 * H].reshape(1, H), zpadb,
         b_ih[2 * H:3 * H].reshape(1, H), zpadb], axis=1)
    bhcat = jnp.concatenate(
        [b_hh[0:H].reshape(1, H), zpadb, b_hh[H:2 * H].reshape(1, H), zpadb,
         b_hh[2 * H:3 * H].reshape(1, H), zpadb], axis=1)

    full = lambda shp: pl.BlockSpec(shp, lambda i: (0,) * len(shp))
    tok, quantized, commit, ctx = pl.pallas_call(
        _vq_fused_kernel,
        grid=(N_BLK,),
        in_specs=[
            pl.BlockSpec((B_BLK, T, D_IN), lambda i: (i, 0, 0)),
            full((D_H1, D_IN)), full((1, D_H1)),
            full((H, D_H1)), full((1, H)),
            full((H, 3 * H)), full((1, H)),
            full((K, H)),
            full((H, G)), full((1, G)),
            full((H, G)), full((1, G)),
        ],
        out_specs=[
            pl.BlockSpec((B_BLK, T), lambda i: (i, 0)),
            pl.BlockSpec((B_BLK, T, H), lambda i: (i, 0, 0)),
            pl.BlockSpec((1, 1, 1), lambda i: (i, 0, 0)),
            pl.BlockSpec((1, 1), lambda i: (0, 0)),
        ],
        out_shape=[
            jax.ShapeDtypeStruct((B, T), jnp.int32),
            jax.ShapeDtypeStruct((B, T, H), jnp.float32),
            jax.ShapeDtypeStruct((N_BLK, 1, 1), jnp.float32),
            jax.ShapeDtypeStruct((1, 1), jnp.float32),
        ],
        scratch_shapes=[
            pltpu.VMEM((T - 1, B, G), jnp.float32),
            pltpu.VMEM((T, B, H), jnp.float32),
        ],
    )(x, W1, b1.reshape(1, D_H1), W2, b2.reshape(1, H), wck,
      bc.reshape(1, H), codebook, wicat, bicat, whcat, bhcat)

    commitment_loss = jnp.sum(commit) / (B * T * H)
    codebook_loss = commitment_loss
    context_loss = ctx[0, 0]
    vq_loss = commitment_loss + 0.25 * codebook_loss + 0.1 * context_loss
    return (tok, quantized, commitment_loss, codebook_loss,
            context_loss, vq_loss)
